# Initial kernel scaffold; baseline (speedup 1.0000x reference)
import jax
import jax.numpy as jnp
from jax.experimental import pallas as pl


def kernel(field_emb, W_align, b_align, ln_g, ln_b, mask_logits, W1, a_src1, a_dst1, b1, W2, a_src2, a_dst2, b2, edge_index, batch_idx):
    raise NotImplementedError("write your pallas kernel here")


# speedup vs baseline: 8.6939x; 8.6939x over previous
import functools
import jax
import jax.numpy as jnp
from jax import lax
from jax.experimental import pallas as pl
from jax.experimental.pallas import tpu as pltpu
from jax.experimental.pallas import tpu_sc as plsc

B = 500
NF = 20
N = B * NF
E = 160000
FD = 64
HID = 128
H1 = 4
HN = N // 2
PW = 256
NTILE = 16
EPT = E // NTILE
KB = 32


def _align_body(fe_ref, wa_ref, ba_ref, g_ref, b_ref, p1_ref, out_ref):
    a = jnp.dot(fe_ref[...], wa_ref[...], preferred_element_type=jnp.float32)
    a = a + ba_ref[...]
    mu = jnp.mean(a, axis=-1, keepdims=True)
    var = jnp.mean((a - mu) ** 2, axis=-1, keepdims=True)
    x = (a - mu) * jax.lax.rsqrt(var + 1e-5) * g_ref[...] + b_ref[...]
    pr = jnp.dot(x, p1_ref[...], preferred_element_type=jnp.float32)
    out_ref[:, 0:HID] = x
    out_ref[:, HID:HID + 8] = pr
    out_ref[:, HID + 8:PW] = jnp.zeros_like(out_ref[:, HID + 8:PW])


def _align(field_emb, W_align, b_align, g_rows, b_rows, P1):
    fe = field_emb.reshape(N, FD)
    R = 2000
    gt = jnp.tile(g_rows, (R // NF, 1))
    bt = jnp.tile(b_rows, (R // NF, 1))
    return pl.pallas_call(
        _align_body,
        grid=(N // R,),
        in_specs=[
            pl.BlockSpec((R, FD), lambda i: (i, 0)),
            pl.BlockSpec((FD, HID), lambda i: (0, 0)),
            pl.BlockSpec((1, HID), lambda i: (0, 0)),
            pl.BlockSpec((R, HID), lambda i: (0, 0)),
            pl.BlockSpec((R, HID), lambda i: (0, 0)),
            pl.BlockSpec((HID, 8), lambda i: (0, 0)),
        ],
        out_specs=pl.BlockSpec((R, PW), lambda i: (i, 0)),
        out_shape=jax.ShapeDtypeStruct((N, PW), jnp.float32),
    )(fe, W_align, b_align.reshape(1, HID), gt, bt, P1)


def _conv1_post_body(y_ref, d_ref, exs_ref, x_ref, w1_ref, b1_ref, p2_ref,
                     out_ref):
    x = x_ref[...]
    z_parts = []
    for h in range(H1):
        num = y_ref[:, h * HID:(h + 1) * HID] + exs_ref[:, h:h + 1] * x
        den = d_ref[:, h:h + 1] + exs_ref[:, h:h + 1]
        seg = num / den
        z_parts.append(jnp.dot(seg, w1_ref[:, h * HID:(h + 1) * HID],
                               preferred_element_type=jnp.float32))
    z = jnp.concatenate(z_parts, axis=1) + b1_ref[...]
    x2 = jnp.where(z > 0, z, jnp.exp(jnp.minimum(z, 0.0)) - 1.0)
    h2 = jnp.dot(x2, p2_ref[:, 0:HID], preferred_element_type=jnp.float32)
    pr = jnp.dot(x2, p2_ref[:, HID:HID + 8],
                 preferred_element_type=jnp.float32)
    out_ref[:, 0:HID] = h2
    out_ref[:, HID:HID + 8] = pr
    out_ref[:, HID + 8:PW] = jnp.zeros_like(out_ref[:, HID + 8:PW])


def _conv1_post(y1, d1, ex1s, x, W1, b1, P2w):
    R = 1000
    return pl.pallas_call(
        _conv1_post_body,
        grid=(N // R,),
        in_specs=[
            pl.BlockSpec((R, H1 * HID), lambda i: (i, 0)),
            pl.BlockSpec((R, H1), lambda i: (i, 0)),
            pl.BlockSpec((R, H1), lambda i: (i, 0)),
            pl.BlockSpec((R, HID), lambda i: (i, 0)),
            pl.BlockSpec((HID, H1 * HID), lambda i: (0, 0)),
            pl.BlockSpec((1, H1 * HID), lambda i: (0, 0)),
            pl.BlockSpec((H1 * HID, HID + 8), lambda i: (0, 0)),
        ],
        out_specs=pl.BlockSpec((R, PW), lambda i: (i, 0)),
        out_shape=jax.ShapeDtypeStruct((N, PW), jnp.float32),
    )(y1, d1, ex1s, x, W1, b1.reshape(1, H1 * HID), P2w)


def _final_body(y_ref, d_ref, exs_ref, h2_ref, b2_ref, out_ref):
    num = y_ref[...] + exs_ref[...] * h2_ref[...]
    den = d_ref[...] + exs_ref[...]
    o = num / den + b2_ref[...]
    g = o.reshape(o.shape[0] // NF, NF, HID)
    out_ref[...] = jnp.mean(g, axis=1)


def _final(y2, d2, ex2s, h2, b2):
    R = N
    return pl.pallas_call(
        _final_body,
        grid=(N // R,),
        in_specs=[
            pl.BlockSpec((R, HID), lambda i: (i, 0)),
            pl.BlockSpec((R, 1), lambda i: (i, 0)),
            pl.BlockSpec((R, 1), lambda i: (i, 0)),
            pl.BlockSpec((R, HID), lambda i: (i, 0)),
            pl.BlockSpec((1, HID), lambda i: (0, 0)),
        ],
        out_specs=pl.BlockSpec((R // NF, HID), lambda i: (i, 0)),
        out_shape=jax.ShapeDtypeStruct((B, HID), jnp.float32),
    )(y2, d2, ex2s, h2, b2.reshape(1, HID))


def _make_edge_kernel(head):
    yrows = -(-HN // 128) * 128
    drows = -(-HN // 128) * 128
    ystripe = yrows // NTILE
    dstripe = drows // NTILE
    mesh = plsc.VectorSubcoreMesh(core_axis_name="c", subcore_axis_name="s",
                                  num_cores=2, num_subcores=NTILE)

    CH = 2000

    def body(src_hbm, dst_hbm, xp_hbm, dp_hbm, zy_hbm, zd_hbm,
             y_out, d_out, sv, dv, cpk, srow, drow, exb,
             msg0, dmsg, srcg, dgidx, sidx0, sidx1, didx,
             yacc, dacc, sem):
        c = lax.axis_index("c")
        s = lax.axis_index("s")
        lo = c * HN
        dummy_dst = lo + HN

        iota16 = lax.iota(jnp.int32, 16)
        ones = jnp.full((16,), 1, jnp.int32)

        def compact_chunk(k, cnt):
            pltpu.sync_copy(src_hbm.at[pl.ds(s * EPT + k * CH, CH)], sv)
            pltpu.sync_copy(dst_hbm.at[pl.ds(s * EPT + k * CH, CH)], dv)
            def compact(i, cnt2):
                s16 = sv[pl.ds(i * 16, 16)]
                d16 = dv[pl.ds(i * 16, 16)]
                m = jnp.logical_and(d16 >= lo, d16 < lo + HN)
                keys = jnp.where(m, 0, 16) + iota16
                pk = s16 * 16384 + d16
                pks = plsc.sort_key_val(keys, pk)[1]
                cpk[pl.ds(cnt2, 16)] = pks
                pc = plsc.all_reduce_population_count(m)
                return cnt2 + pc[0]
            return lax.fori_loop(0, CH // 16, compact, cnt)
        M = lax.fori_loop(0, EPT // CH, compact_chunk, jnp.int32(0))
        cpk[pl.ds(M, 16)] = ones * dummy_dst
        cpk[pl.ds(M + 16, 16)] = ones * dummy_dst
        nb = (M + (KB - 1)) // KB

        for p in range(1):
            pltpu.sync_copy(zy_hbm, yacc.at[pl.ds(s * ystripe, ystripe)])
            pltpu.sync_copy(zd_hbm, dacc.at[pl.ds(s * dstripe, dstripe)])
            plsc.subcore_barrier()

            def block(b, _):
                for j in range(KB // 16):
                    p16 = cpk[pl.ds(b * KB + j * 16, 16)]
                    d16 = jnp.bitwise_and(p16, 16383)
                    s16 = lax.shift_right_logical(p16, 14)
                    srcg[pl.ds(j * 16, 16)] = s16
                    dgidx[pl.ds(j * 16, 16)] = d16
                    dl = d16 - lo
                    didx[pl.ds(j * 16, 16)] = dl
                    sidx0[pl.ds(j * 16, 16)] = dl
                pltpu.async_copy(xp_hbm.at[srcg], srow, sem).wait()
                pltpu.async_copy(dp_hbm.at[dgidx], drow, sem).wait()
                for j in range(KB // 16):
                    rows = j * 16 + iota16
                    for h2 in range(1):
                        h = head
                        as_v = plsc.load_gather(
                            srow, [rows, jnp.full((16,), HID + h, jnp.int32)])
                        ad_v = plsc.load_gather(
                            drow, [rows, jnp.full((16,), h, jnp.int32)])
                        mt_v = plsc.load_gather(
                            drow, [rows, jnp.full((16,), 4 + h, jnp.int32)])
                        z = as_v + ad_v
                        e = jnp.where(z >= 0, z, 0.2 * z)
                        ex = jnp.exp(e - mt_v)
                        exb[h2, pl.ds(j * 16, 16)] = ex
                        plsc.store_scatter(
                            dmsg, [rows, jnp.full((16,), h2, jnp.int32)], ex)
                def scale(g, _):
                    exv0 = exb[0, pl.ds(g * 16, 16)]
                    for l in range(16):
                        row = g * 16 + l
                        x0 = exv0[l]
                        for j in range(HID // 16):
                            v = srow[row, pl.ds(j * 16, 16)]
                            msg0[row, pl.ds(j * 16, 16)] = v * x0
                    return 0
                lax.fori_loop(0, KB // 16, scale, 0)
                pltpu.sync_copy(msg0, yacc.at[sidx0], add=True)
                pltpu.sync_copy(dmsg, dacc.at[didx], add=True)
                return 0
            lax.fori_loop(0, nb, block, 0)
            plsc.subcore_barrier()

            pltpu.sync_copy(yacc.at[pl.ds(s * ystripe, ystripe)],
                            y_out.at[p, c, pl.ds(s * ystripe, ystripe)])
            pltpu.sync_copy(dacc.at[pl.ds(s * dstripe, dstripe)],
                            d_out.at[p, c, pl.ds(s * dstripe, dstripe)])
            plsc.subcore_barrier()

    kern = pl.kernel(
        body,
        compiler_params=pltpu.CompilerParams(needs_layout_passes=False),
        out_type=(
            jax.ShapeDtypeStruct((1, 2, yrows, HID), jnp.float32),
            jax.ShapeDtypeStruct((1, 2, drows, 16), jnp.float32),
        ),
        mesh=mesh,
        scratch_types=[
            pltpu.VMEM((CH,), jnp.int32),
            pltpu.VMEM((CH,), jnp.int32),
            pltpu.VMEM((EPT + 2 * NTILE,), jnp.int32),
            pltpu.VMEM((KB, PW), jnp.float32),
            pltpu.VMEM((KB, 128), jnp.float32),
            pltpu.VMEM((2, KB), jnp.float32),
            pltpu.VMEM((KB, HID), jnp.float32),
            pltpu.VMEM((KB, 16), jnp.float32),
            pltpu.VMEM((KB,), jnp.int32),
            pltpu.VMEM((KB,), jnp.int32),
            pltpu.VMEM((KB,), jnp.int32),
            pltpu.VMEM((KB,), jnp.int32),
            pltpu.VMEM((KB,), jnp.int32),
            pltpu.VMEM_SHARED((yrows, HID), jnp.float32),
            pltpu.VMEM_SHARED((drows, 16), jnp.float32),
            pltpu.SemaphoreType.DMA,
        ],
    )
    return kern


@functools.lru_cache(maxsize=None)
def _get_edge_kernel(head):
    return _make_edge_kernel(head)


def _leaky(v):
    return jnp.where(v >= 0, v, 0.2 * v)


def kernel(field_emb, W_align, b_align, ln_g, ln_b, mask_logits,
           W1, a_src1, a_dst1, b1, W2, a_src2, a_dst2, b2,
           edge_index, batch_idx):
    gate = jax.nn.sigmoid(mask_logits)
    g_rows = gate[:, None] * ln_g[None, :]
    b_rows = gate[:, None] * ln_b[None, :]

    W1r = W1.reshape(HID, H1, HID)
    Ps1 = jnp.einsum("ihc,hc->ih", W1r, a_src1)
    Pd1 = jnp.einsum("ihc,hc->ih", W1r, a_dst1)
    P1 = jnp.concatenate([Ps1, Pd1], axis=1)

    xp = _align(field_emb, W_align, b_align, g_rows, b_rows, P1)
    x = xp[:, 0:HID]
    as1 = xp[:, HID:HID + H1]
    ad1 = xp[:, HID + H1:HID + 2 * H1]

    Mas1 = jnp.max(as1, axis=0)
    mt1 = _leaky(Mas1[None, :] + ad1)
    ex1s = jnp.exp(_leaky(as1 + ad1) - mt1)
    dp1 = jnp.zeros((N + NTILE, 128), jnp.float32)
    dp1 = dp1.at[:N, 0:H1].set(ad1).at[:N, 4:4 + H1].set(mt1)

    src = edge_index[0]
    dst = edge_index[1]
    zy = jnp.zeros((-(-HN // 128) * 128 // NTILE, HID), jnp.float32)
    zd = jnp.zeros((-(-HN // 128) * 128 // NTILE, 16), jnp.float32)

    y1hs, d1hs = [], []
    chain = jnp.zeros((), jnp.float32)
    for h in range(H1):
        zy_h = zy + chain
        yho, dho = _get_edge_kernel(h)(src, dst, xp, dp1, zy_h, zd)
        chain = yho[0, 0, HN, 0] * 0.0
        y1hs.append(yho[0, :, :HN, :].reshape(N, HID))
        d1hs.append(dho[0, :, :HN, 0].reshape(N))
    y1 = jnp.concatenate(y1hs, axis=1)
    d1 = jnp.stack(d1hs, axis=1)

    P2s = W2 @ a_src2[0]
    P2d = W2 @ a_dst2[0]
    P2w = jnp.concatenate(
        [W2, jnp.stack([P2s, P2d], axis=1),
         jnp.zeros((H1 * HID, 6), jnp.float32)], axis=1)
    P2w = jnp.pad(P2w, ((0, 0), (0, HID + 8 - P2w.shape[1])))

    xp2 = _conv1_post(y1, d1, ex1s, x, W1, b1, P2w)
    h2 = xp2[:, 0:HID]
    as2 = xp2[:, HID]
    ad2 = xp2[:, HID + 1]

    Mas2 = jnp.max(as2)
    mt2 = _leaky(Mas2 + ad2)
    ex2s = jnp.exp(_leaky(as2 + ad2) - mt2)
    dp2 = jnp.zeros((N + NTILE, 128), jnp.float32)
    dp2 = dp2.at[:N, 0].set(ad2).at[:N, 4].set(mt2)
    xp2sc = xp2

    y2o, d2o = _get_edge_kernel(0)(src, dst, xp2sc, dp2, zy + chain, zd)
    y2 = y2o[0, :, :HN, :].reshape(N, HID)
    d2 = d2o[0, :, :HN, 0].reshape(N, 1)

    graph_embedding = _final(y2, d2, ex2s[:, None], h2, b2)
    return (graph_embedding, gate)
